# baseline (speedup 1.0000x reference)
import functools

import jax
import jax.numpy as jnp
from jax import lax
from jax.experimental import pallas as pl
from jax.experimental.pallas import tpu as pltpu
from jax.experimental.pallas import tpu_sc as plsc

_B = 16384
_V = 1_000_000
_F = 32
_NG = 1954
_PR = _NG * 128

_NC = 2
_NS = 16
_NW = _NC * _NS
_BPW = _B // _NW
_CH = 256
_NCHK = _BPW // _CH


def _repack_body(ut, it, ou, oi):
    for src, dst in ((ut, ou), (it, oi)):
        x = src[...]
        dst[...] = jnp.concatenate(
            [x[:, 128 * a:128 * (a + 1)].T for a in range(4)], axis=1)


def _repack(uT, iT):
    return pl.pallas_call(
        _repack_body,
        grid=(_NG,),
        in_specs=[pl.BlockSpec((_F, 512), lambda g: (0, g))] * 2,
        out_specs=[pl.BlockSpec((128, 128), lambda g: (g, 0))] * 2,
        out_shape=[jax.ShapeDtypeStruct((_PR, 128), jnp.float32)] * 2,
    )(uT, iT)


def _sc_gather_body(r_u, r_i, u_idx, i_idx, j_idx,
                    out_u, out_i, out_j,
                    idxu, idxi, idxj, su, si, sj, sem):
    wid = lax.axis_index("s") * _NC + lax.axis_index("c")
    base = wid * _BPW
    pltpu.sync_copy(u_idx.at[pl.ds(base, _BPW)], idxu)
    pltpu.sync_copy(i_idx.at[pl.ds(base, _BPW)], idxi)
    pltpu.sync_copy(j_idx.at[pl.ds(base, _BPW)], idxj)

    for c in range(_NCHK):
        def issue(g, carry):
            off = c * _CH + g * 16
            gu = ((idxu[pl.ds(off, 16)] >> 9) << 7) | (idxu[pl.ds(off, 16)] & 127)
            gi = ((idxi[pl.ds(off, 16)] >> 9) << 7) | (idxi[pl.ds(off, 16)] & 127)
            gj = ((idxj[pl.ds(off, 16)] >> 9) << 7) | (idxj[pl.ds(off, 16)] & 127)
            for k in range(16):
                r = g * 16 + k
                pltpu.async_copy(r_u.at[pl.ds(gu[k], 1), :],
                                 su.at[pl.ds(r, 1), :], sem)
                pltpu.async_copy(r_i.at[pl.ds(gi[k], 1), :],
                                 si.at[pl.ds(r, 1), :], sem)
                pltpu.async_copy(r_i.at[pl.ds(gj[k], 1), :],
                                 sj.at[pl.ds(r, 1), :], sem)
            return carry

        lax.fori_loop(0, _CH // 16, issue, 0)

        def drain(r, carry):
            for _ in range(3):
                pltpu.make_async_copy(
                    r_u.at[pl.ds(0, 1), :],
                    su.at[pl.ds(0, 1), :], sem).wait()
            return carry

        lax.fori_loop(0, _CH, drain, 0)
        pltpu.sync_copy(su, out_u.at[pl.ds(base + c * _CH, _CH)])
        pltpu.sync_copy(si, out_i.at[pl.ds(base + c * _CH, _CH)])
        pltpu.sync_copy(sj, out_j.at[pl.ds(base + c * _CH, _CH)])


@functools.cache
def _sc_gather():
    return pl.kernel(
        _sc_gather_body,
        mesh=plsc.VectorSubcoreMesh(
            core_axis_name="c", subcore_axis_name="s", num_cores=_NC),
        out_type=[jax.ShapeDtypeStruct((_B, 128), jnp.float32)] * 3,
        scratch_types=[
            pltpu.VMEM((_BPW,), jnp.int32),
            pltpu.VMEM((_BPW,), jnp.int32),
            pltpu.VMEM((_BPW,), jnp.int32),
            pltpu.VMEM((_CH, 128), jnp.float32),
            pltpu.VMEM((_CH, 128), jnp.float32),
            pltpu.VMEM((_CH, 128), jnp.float32),
            pltpu.SemaphoreType.DMA,
        ],
    )


_BLK = 2048


def _select(slab, idx):
    a = (idx >> 7) & 3
    e = jnp.zeros((slab.shape[0], _F), jnp.float32)
    for q in range(4):
        e += slab[:, _F * q:_F * (q + 1)] * (a == q).astype(jnp.float32)
    return e


def _mlp_body(su, si, sj, ui, ii, ji, w1u, w1i, b1, w2, b2, w3, b3, wf, bfr,
              oi, oj):
    eu = _select(su[...], ui[...])
    hu = jnp.dot(eu, w1u[...], preferred_element_type=jnp.float32)

    def head(slab, idx, o_ref):
        e = _select(slab[...], idx[...])
        h = jax.nn.relu(hu + jnp.dot(e, w1i[...],
                                     preferred_element_type=jnp.float32)
                        + b1[...])
        h = jax.nn.relu(jnp.dot(h, w2[...],
                                preferred_element_type=jnp.float32) + b2[...])
        h = jax.nn.relu(jnp.dot(h, w3[...],
                                preferred_element_type=jnp.float32) + b3[...])
        o_ref[...] = jnp.sum(h * wf[...], axis=1) + bfr[0, 0]

    head(si, ii, oi)
    head(sj, ji, oj)


def kernel(user, item_i, item_j, context, uEmbd, iEmbd,
           W1, b1, W2, b2, W3, b3, Wf, bf):
    del context
    user = user.astype(jnp.int32)
    item_i = item_i.astype(jnp.int32)
    item_j = item_j.astype(jnp.int32)
    r_u, r_i = _repack(uEmbd.T, iEmbd.T)
    su, si, sj = _sc_gather()(r_u, r_i, user, item_i, item_j)

    grid = (_B // _BLK,)
    slab_spec = pl.BlockSpec((_BLK, 128), lambda i: (i, 0))
    idx_spec = pl.BlockSpec((_BLK, 1), lambda i: (i, 0))
    full2 = lambda shp: pl.BlockSpec(shp, lambda i: (0, 0))
    out_spec = pl.BlockSpec((_BLK,), lambda i: (i,))
    pred_i, pred_j = pl.pallas_call(
        _mlp_body,
        grid=grid,
        in_specs=[slab_spec, slab_spec, slab_spec,
                  idx_spec, idx_spec, idx_spec,
                  full2((_F, 32)), full2((_F, 32)),
                  full2((1, 32)),
                  full2((32, 16)), full2((1, 16)),
                  full2((16, 8)), full2((1, 8)),
                  full2((1, 8)), full2((1, 1))],
        out_specs=[out_spec, out_spec],
        out_shape=[jax.ShapeDtypeStruct((_B,), jnp.float32)] * 2,
    )(su, si, sj,
      user.reshape(_B, 1), item_i.reshape(_B, 1), item_j.reshape(_B, 1),
      W1[:_F, :], W1[_F:, :], b1.reshape(1, 32),
      W2, b2.reshape(1, 16), W3, b3.reshape(1, 8),
      Wf.reshape(1, 8), bf.reshape(1, 1))
    return (pred_i, pred_j)


# speedup vs baseline: 1.4273x; 1.4273x over previous
import functools

import jax
import jax.numpy as jnp
from jax import lax
from jax.experimental import pallas as pl
from jax.experimental.pallas import tpu as pltpu
from jax.experimental.pallas import tpu_sc as plsc

_B = 16384
_V = 1_000_000
_F = 32

_NC = 2
_NS = 16
_NW = _NC * _NS
_BPW = _B // _NW
_CH = 256
_NCHK = _BPW // _CH


def _sc_gather_body(r_u, r_i, u_idx, i_idx, j_idx,
                    out_u, out_i, out_j,
                    idxu, idxi, idxj, su, si, sj, sem):
    wid = lax.axis_index("s") * _NC + lax.axis_index("c")
    base = wid * _BPW
    pltpu.sync_copy(u_idx.at[pl.ds(base, _BPW)], idxu)
    pltpu.sync_copy(i_idx.at[pl.ds(base, _BPW)], idxi)
    pltpu.sync_copy(j_idx.at[pl.ds(base, _BPW)], idxj)

    for c in range(_NCHK):
        def issue(g, carry):
            off = c * _CH + g * 16
            gu = idxu[pl.ds(off, 16)] >> 2
            gi = idxi[pl.ds(off, 16)] >> 2
            gj = idxj[pl.ds(off, 16)] >> 2
            for k in range(16):
                r = g * 16 + k
                pltpu.async_copy(r_u.at[pl.ds(gu[k], 1), :],
                                 su.at[pl.ds(r, 1), :], sem)
                pltpu.async_copy(r_i.at[pl.ds(gi[k], 1), :],
                                 si.at[pl.ds(r, 1), :], sem)
                pltpu.async_copy(r_i.at[pl.ds(gj[k], 1), :],
                                 sj.at[pl.ds(r, 1), :], sem)
            return carry

        lax.fori_loop(0, _CH // 16, issue, 0)

        def drain(r, carry):
            for _ in range(3):
                pltpu.make_async_copy(
                    r_u.at[pl.ds(0, 1), :],
                    su.at[pl.ds(0, 1), :], sem).wait()
            return carry

        lax.fori_loop(0, _CH, drain, 0)
        pltpu.sync_copy(su, out_u.at[pl.ds(base + c * _CH, _CH)])
        pltpu.sync_copy(si, out_i.at[pl.ds(base + c * _CH, _CH)])
        pltpu.sync_copy(sj, out_j.at[pl.ds(base + c * _CH, _CH)])


@functools.cache
def _sc_gather():
    return pl.kernel(
        _sc_gather_body,
        mesh=plsc.VectorSubcoreMesh(
            core_axis_name="c", subcore_axis_name="s", num_cores=_NC),
        out_type=[jax.ShapeDtypeStruct((_B, 128), jnp.float32)] * 3,
        scratch_types=[
            pltpu.VMEM((_BPW,), jnp.int32),
            pltpu.VMEM((_BPW,), jnp.int32),
            pltpu.VMEM((_BPW,), jnp.int32),
            pltpu.VMEM((_CH, 128), jnp.float32),
            pltpu.VMEM((_CH, 128), jnp.float32),
            pltpu.VMEM((_CH, 128), jnp.float32),
            pltpu.SemaphoreType.DMA,
        ],
    )


_BLK = 2048


def _select(slab, idx):
    a = idx & 3
    e = jnp.zeros((slab.shape[0], _F), jnp.float32)
    for q in range(4):
        e += slab[:, _F * q:_F * (q + 1)] * (a == q).astype(jnp.float32)
    return e


def _mlp_body(su, si, sj, ui, ii, ji, w1u, w1i, b1, w2, b2, w3, b3, wf, bfr,
              oi, oj):
    eu = _select(su[...], ui[...])
    hu = jnp.dot(eu, w1u[...], preferred_element_type=jnp.float32)

    def head(slab, idx, o_ref):
        e = _select(slab[...], idx[...])
        h = jax.nn.relu(hu + jnp.dot(e, w1i[...],
                                     preferred_element_type=jnp.float32)
                        + b1[...])
        h = jax.nn.relu(jnp.dot(h, w2[...],
                                preferred_element_type=jnp.float32) + b2[...])
        h = jax.nn.relu(jnp.dot(h, w3[...],
                                preferred_element_type=jnp.float32) + b3[...])
        o_ref[...] = jnp.sum(h * wf[...], axis=1) + bfr[0, 0]

    head(si, ii, oi)
    head(sj, ji, oj)


def kernel(user, item_i, item_j, context, uEmbd, iEmbd,
           W1, b1, W2, b2, W3, b3, Wf, bf):
    del context
    user = user.astype(jnp.int32)
    item_i = item_i.astype(jnp.int32)
    item_j = item_j.astype(jnp.int32)
    r_u = uEmbd.reshape(_V // 4, 128)
    r_i = iEmbd.reshape(_V // 4, 128)
    su, si, sj = _sc_gather()(r_u, r_i, user, item_i, item_j)

    grid = (_B // _BLK,)
    slab_spec = pl.BlockSpec((_BLK, 128), lambda i: (i, 0))
    idx_spec = pl.BlockSpec((_BLK, 1), lambda i: (i, 0))
    full2 = lambda shp: pl.BlockSpec(shp, lambda i: (0, 0))
    out_spec = pl.BlockSpec((_BLK,), lambda i: (i,))
    pred_i, pred_j = pl.pallas_call(
        _mlp_body,
        grid=grid,
        in_specs=[slab_spec, slab_spec, slab_spec,
                  idx_spec, idx_spec, idx_spec,
                  full2((_F, 32)), full2((_F, 32)),
                  full2((1, 32)),
                  full2((32, 16)), full2((1, 16)),
                  full2((16, 8)), full2((1, 8)),
                  full2((1, 8)), full2((1, 1))],
        out_specs=[out_spec, out_spec],
        out_shape=[jax.ShapeDtypeStruct((_B,), jnp.float32)] * 2,
    )(su, si, sj,
      user.reshape(_B, 1), item_i.reshape(_B, 1), item_j.reshape(_B, 1),
      W1[:_F, :], W1[_F:, :], b1.reshape(1, 32),
      W2, b2.reshape(1, 16), W3, b3.reshape(1, 8),
      Wf.reshape(1, 8), bf.reshape(1, 1))
    return (pred_i, pred_j)
